# Initial kernel scaffold; baseline (speedup 1.0000x reference)
import jax
import jax.numpy as jnp
from jax.experimental import pallas as pl


def kernel(x, y, directions):
    raise NotImplementedError("write your pallas kernel here")


# speedup vs baseline: 26.8826x; 26.8826x over previous
import jax
import jax.numpy as jnp
from jax.experimental import pallas as pl


def kernel(x, y, directions):
    b, c, h, w = x.shape
    norm = jnp.sqrt(jnp.sum(jnp.square(directions), axis=-1, keepdims=True))
    dirs = directions / norm
    px = jnp.matmul(dirs, x.reshape(c, h * w))
    py = jnp.matmul(dirs, y.reshape(c, h * w))
    u = jnp.sort(px, axis=1)
    v = jnp.sort(py, axis=1)
    return jnp.mean(jnp.abs(u - v))


# speedup vs baseline: 376.2660x; 13.9966x over previous
import functools

import jax
import jax.numpy as jnp
from jax import lax
from jax.experimental import pallas as pl
from jax.experimental.pallas import tpu as pltpu
from jax.experimental.pallas import tpu_sc as plsc

NDIR = 96
NPTS = 512 * 512
KBINS = 65536
NC, NS, L = 2, 16, 16
ROWS_PER_CORE = NDIR // NC
PTS_PER_SUB = NPTS // NS
CHUNK = 128
NCHUNK = PTS_PER_SUB // CHUNK
BINS_PER_SUB = KBINS // NS
BN = 2048
GRID = NPTS // BN


def _tc_body(dirs_ref, xr_ref, yr_ref, px_ref, py_ref, mn_ref, mx_ref):
    d = dirs_ref[...]
    pxb = jnp.dot(d, xr_ref[...], preferred_element_type=jnp.float32)
    pyb = jnp.dot(d, yr_ref[...], preferred_element_type=jnp.float32)
    px_ref[...] = pxb
    py_ref[...] = pyb
    bmin = jnp.minimum(jnp.min(pxb, axis=1, keepdims=True),
                       jnp.min(pyb, axis=1, keepdims=True))
    bmax = jnp.maximum(jnp.max(pxb, axis=1, keepdims=True),
                       jnp.max(pyb, axis=1, keepdims=True))
    bmin = jnp.broadcast_to(bmin, (NDIR, 128))
    bmax = jnp.broadcast_to(bmax, (NDIR, 128))
    j = pl.program_id(0)
    mn_ref[...] = jnp.where(j == 0, bmin, jnp.minimum(mn_ref[...], bmin))
    mx_ref[...] = jnp.where(j == 0, bmax, jnp.maximum(mx_ref[...], bmax))


def _project(dirs, xr, yr):
    return pl.pallas_call(
        _tc_body,
        grid=(GRID,),
        in_specs=[
            pl.BlockSpec((NDIR, NDIR), lambda j: (0, 0)),
            pl.BlockSpec((NDIR, BN), lambda j: (0, j)),
            pl.BlockSpec((NDIR, BN), lambda j: (0, j)),
        ],
        out_specs=[
            pl.BlockSpec((NDIR, BN), lambda j: (0, j)),
            pl.BlockSpec((NDIR, BN), lambda j: (0, j)),
            pl.BlockSpec((NDIR, 128), lambda j: (0, 0)),
            pl.BlockSpec((NDIR, 128), lambda j: (0, 0)),
        ],
        out_shape=[
            jax.ShapeDtypeStruct((NDIR, NPTS), jnp.float32),
            jax.ShapeDtypeStruct((NDIR, NPTS), jnp.float32),
            jax.ShapeDtypeStruct((NDIR, 128), jnp.float32),
            jax.ShapeDtypeStruct((NDIR, 128), jnp.float32),
        ],
        compiler_params=pltpu.CompilerParams(
            dimension_semantics=("arbitrary",)),
    )(dirs, xr, yr)


def _sc_body(px, py, lo_h, w_h, invw_h, out_h,
             lo_v, w_v, invw_v, ptbuf, idxbuf, valbuf, ones_r, negones_r,
             zeros_r, d_chunk, s_chunk, tmp16, iota_r, totv16, partials,
             d_sh, s_sh, totals_sh):
    c = lax.axis_index("c")
    s = lax.axis_index("s")
    iota = lax.iota(jnp.int32, L)

    iota_r[...] = iota

    def _init(i, _):
        ones_r[pl.ds(i * L, L)] = jnp.full((L,), 1.0, jnp.float32)
        negones_r[pl.ds(i * L, L)] = jnp.full((L,), -1.0, jnp.float32)
        return 0
    lax.fori_loop(0, CHUNK // L, _init, 0)

    def _zinit(i, _):
        zeros_r[pl.ds(i * L, L)] = jnp.zeros((L,), jnp.float32)
        return 0
    lax.fori_loop(0, BINS_PER_SUB // L, _zinit, 0)

    pltpu.sync_copy(lo_h, lo_v)
    pltpu.sync_copy(w_h, w_v)
    pltpu.sync_copy(invw_h, invw_v)

    def _bcast(table_ref, r):
        acc = jnp.float32(0.0)
        for kk in range(128 // L):
            vec = table_ref[pl.ds(kk * L, L)]
            acc = acc + jnp.sum(jnp.where(iota + kk * L == r, vec, 0.0))
        return jnp.full((L,), acc, jnp.float32)

    def _row(j, _):
        r = c * ROWS_PER_CORE + j
        lo = _bcast(lo_v, r)
        wv = _bcast(w_v, r)
        invw = _bcast(invw_v, r)

        pltpu.sync_copy(zeros_r, d_sh.at[pl.ds(s * BINS_PER_SUB, BINS_PER_SUB)])
        pltpu.sync_copy(zeros_r, s_sh.at[pl.ds(s * BINS_PER_SUB, BINS_PER_SUB)])

        @pl.when(s == 0)
        def _():
            pltpu.sync_copy(zeros_r.at[pl.ds(0, L)], totals_sh)
        plsc.subcore_barrier()

        for side in (0, 1):
            src = px if side == 0 else py
            sgn = 1.0 if side == 0 else -1.0
            cnt_ref = ones_r if side == 0 else negones_r
            pltpu.sync_copy(src.at[r, pl.ds(s * PTS_PER_SUB, PTS_PER_SUB)],
                            ptbuf)

            def _chunk(i, _):
                def _vreg(k, _):
                    t = ptbuf[pl.ds(i * CHUNK + k * L, L)]
                    f = (t - lo) * invw
                    f = jnp.minimum(f, float(KBINS - 1))
                    f = jnp.maximum(f, 0.0)
                    bi = f.astype(jnp.int32)
                    er = lo + (bi.astype(jnp.float32) + 1.0) * wv
                    idxbuf[i, pl.ds(k * L, L)] = bi
                    valbuf[i, pl.ds(k * L, L)] = sgn * (er - t)
                    return 0
                lax.fori_loop(0, CHUNK // L, _vreg, 0)
                pltpu.sync_copy(cnt_ref, d_sh.at[idxbuf.at[i]], add=True)
                pltpu.sync_copy(valbuf.at[i], s_sh.at[idxbuf.at[i]], add=True)
                return 0
            lax.fori_loop(0, NCHUNK, _chunk, 0)
        plsc.subcore_barrier()

        pltpu.sync_copy(d_sh.at[pl.ds(s * BINS_PER_SUB, BINS_PER_SUB)], d_chunk)
        pltpu.sync_copy(s_sh.at[pl.ds(s * BINS_PER_SUB, BINS_PER_SUB)], s_chunk)

        def _csum(t, acc):
            return acc + d_chunk[pl.ds(t * L, L)]
        chunk_sum_v = lax.fori_loop(0, BINS_PER_SUB // L, _csum,
                                    jnp.zeros((L,), jnp.float32))
        tt = jnp.sum(chunk_sum_v)
        tmp16[...] = jnp.where(iota > s, tt, 0.0)
        pltpu.sync_copy(tmp16, totals_sh.at[iota_r], add=True)
        plsc.subcore_barrier()
        pltpu.sync_copy(totals_sh, totv16)
        carry0 = jnp.sum(jnp.where(iota == s, totv16[...], 0.0))

        def _scan(t, st):
            carry, acc = st
            d = d_chunk[pl.ds(t * L, L)]
            sv = s_chunk[pl.ds(t * L, L)]
            pc = plsc.cumsum(d)
            c_left = carry + (pc - d)
            acc = acc + jnp.abs(c_left * wv + sv)
            return (carry + jnp.sum(d), acc)
        _, accv = lax.fori_loop(0, BINS_PER_SUB // L, _scan,
                                (carry0, jnp.zeros((L,), jnp.float32)))
        partials[pl.ds(j * L, L)] = accv
        plsc.subcore_barrier()
        return 0

    lax.fori_loop(0, ROWS_PER_CORE, _row, 0)
    pltpu.sync_copy(partials, out_h.at[c, s])


def _sc_cramer(px, py, lo, w, invw):
    mesh = plsc.VectorSubcoreMesh(core_axis_name="c", subcore_axis_name="s")
    f = pl.kernel(
        _sc_body,
        out_type=jax.ShapeDtypeStruct((NC, NS, ROWS_PER_CORE * L), jnp.float32),
        mesh=mesh,
        scratch_types=[
            pltpu.VMEM((128,), jnp.float32),
            pltpu.VMEM((128,), jnp.float32),
            pltpu.VMEM((128,), jnp.float32),
            pltpu.VMEM((PTS_PER_SUB,), jnp.float32),
            pltpu.VMEM((NCHUNK, CHUNK), jnp.int32),
            pltpu.VMEM((NCHUNK, CHUNK), jnp.float32),
            pltpu.VMEM((CHUNK,), jnp.float32),
            pltpu.VMEM((CHUNK,), jnp.float32),
            pltpu.VMEM((BINS_PER_SUB,), jnp.float32),
            pltpu.VMEM((BINS_PER_SUB,), jnp.float32),
            pltpu.VMEM((BINS_PER_SUB,), jnp.float32),
            pltpu.VMEM((L,), jnp.float32),
            pltpu.VMEM((L,), jnp.int32),
            pltpu.VMEM((L,), jnp.float32),
            pltpu.VMEM((ROWS_PER_CORE * L,), jnp.float32),
            pltpu.VMEM_SHARED((KBINS,), jnp.float32),
            pltpu.VMEM_SHARED((KBINS,), jnp.float32),
            pltpu.VMEM_SHARED((L,), jnp.float32),
        ],
        compiler_params=pltpu.CompilerParams(needs_layout_passes=False),
    )
    return f(px, py, lo, w, invw)


def kernel(x, y, directions):
    b, ch, h, w_ = x.shape
    n = h * w_
    norm = jnp.sqrt(jnp.sum(jnp.square(directions), axis=-1, keepdims=True))
    dirs = directions / norm
    xr = x.reshape(ch, n)
    yr = y.reshape(ch, n)
    px, py, mn, mx = _project(dirs, xr, yr)
    lo = jnp.min(mn, axis=1)
    hi = jnp.max(mx, axis=1)
    wbin = jnp.maximum(hi - lo, 1e-30) / KBINS
    pad = jnp.zeros((128 - NDIR,), jnp.float32)
    lo_p = jnp.concatenate([lo, pad])
    w_p = jnp.concatenate([wbin, pad + 1.0])
    invw_p = jnp.concatenate([1.0 / wbin, pad + 1.0])
    out = _sc_cramer(px, py, lo_p, w_p, invw_p)
    return jnp.sum(out) / (NDIR * n)


# speedup vs baseline: 560.8568x; 1.4906x over previous
import functools

import jax
import jax.numpy as jnp
from jax import lax
from jax.experimental import pallas as pl
from jax.experimental.pallas import tpu as pltpu
from jax.experimental.pallas import tpu_sc as plsc

NDIR = 96
NPTS = 512 * 512
KBINS = 32768
NC, NS, L = 2, 16, 16
ROWS_PER_CORE = NDIR // NC
PTS_PER_SUB = NPTS // NS
CHUNK = 128
NCHUNK = PTS_PER_SUB // CHUNK
BINS_PER_SUB = KBINS // NS
BN = 2048
GRID = NPTS // BN
SCAT_WIN = 8


def _tc_body(dirs_ref, xr_ref, yr_ref, px_ref, py_ref, mn_ref, mx_ref):
    d = dirs_ref[...]
    pxb = jnp.dot(d, xr_ref[...], preferred_element_type=jnp.float32)
    pyb = jnp.dot(d, yr_ref[...], preferred_element_type=jnp.float32)
    px_ref[...] = pxb
    py_ref[...] = pyb
    bmin = jnp.minimum(jnp.min(pxb, axis=1, keepdims=True),
                       jnp.min(pyb, axis=1, keepdims=True))
    bmax = jnp.maximum(jnp.max(pxb, axis=1, keepdims=True),
                       jnp.max(pyb, axis=1, keepdims=True))
    bmin = jnp.broadcast_to(bmin, (NDIR, 128))
    bmax = jnp.broadcast_to(bmax, (NDIR, 128))
    j = pl.program_id(0)
    mn_ref[...] = jnp.where(j == 0, bmin, jnp.minimum(mn_ref[...], bmin))
    mx_ref[...] = jnp.where(j == 0, bmax, jnp.maximum(mx_ref[...], bmax))


def _project(dirs, xr, yr):
    return pl.pallas_call(
        _tc_body,
        grid=(GRID,),
        in_specs=[
            pl.BlockSpec((NDIR, NDIR), lambda j: (0, 0)),
            pl.BlockSpec((NDIR, BN), lambda j: (0, j)),
            pl.BlockSpec((NDIR, BN), lambda j: (0, j)),
        ],
        out_specs=[
            pl.BlockSpec((NDIR, BN), lambda j: (0, j)),
            pl.BlockSpec((NDIR, BN), lambda j: (0, j)),
            pl.BlockSpec((NDIR, 128), lambda j: (0, 0)),
            pl.BlockSpec((NDIR, 128), lambda j: (0, 0)),
        ],
        out_shape=[
            jax.ShapeDtypeStruct((NDIR, NPTS), jnp.float32),
            jax.ShapeDtypeStruct((NDIR, NPTS), jnp.float32),
            jax.ShapeDtypeStruct((NDIR, 128), jnp.float32),
            jax.ShapeDtypeStruct((NDIR, 128), jnp.float32),
        ],
        compiler_params=pltpu.CompilerParams(
            dimension_semantics=("arbitrary",)),
    )(dirs, xr, yr)


def _sc_body(px, py, lo_h, w_h, invw_h, out_h,
             lo_v, w_v, invw_v, ptbuf, idxbuf, pvalbuf,
             zeros_i, t_chunk, tmp16, iota_r, totv16, partials,
             tab_sh, totals_sh, sem):
    c = lax.axis_index("c")
    s = lax.axis_index("s")
    iota = lax.iota(jnp.int32, L)

    iota_r[...] = iota

    def _zinit(i, _):
        zeros_i[pl.ds(i * L, L)] = jnp.zeros((L,), jnp.int32)
        return 0
    lax.fori_loop(0, BINS_PER_SUB // L, _zinit, 0)

    pltpu.sync_copy(lo_h, lo_v)
    pltpu.sync_copy(w_h, w_v)
    pltpu.sync_copy(invw_h, invw_v)

    def _bcast(table_ref, r):
        acc = jnp.float32(0.0)
        for kk in range(128 // L):
            vec = table_ref[pl.ds(kk * L, L)]
            acc = acc + jnp.sum(jnp.where(iota + kk * L == r, vec, 0.0))
        return jnp.full((L,), acc, jnp.float32)

    def _row(j, _):
        r = c * ROWS_PER_CORE + j
        lo = _bcast(lo_v, r)
        wv = _bcast(w_v, r)
        invw = _bcast(invw_v, r)

        pltpu.sync_copy(zeros_i, tab_sh.at[pl.ds(s * BINS_PER_SUB, BINS_PER_SUB)])

        @pl.when(s == 0)
        def _():
            tmp16[...] = jnp.zeros((L,), jnp.float32)
            pltpu.sync_copy(tmp16, totals_sh)
        plsc.subcore_barrier()

        for side in (0, 1):
            src = px if side == 0 else py
            pltpu.sync_copy(src.at[r, pl.ds(s * PTS_PER_SUB, PTS_PER_SUB)],
                            ptbuf)

            def _chunk(i, _):
                def _vreg(k, _):
                    t = ptbuf[pl.ds(i * CHUNK + k * L, L)]
                    f = (t - lo) * invw
                    bi = jnp.minimum(f, float(KBINS - 1)).astype(jnp.int32)
                    q = (((bi + 1).astype(jnp.float32) - f) * 1024.0
                         + 0.5).astype(jnp.int32)
                    pv = q + (1 << 19)
                    if side == 1:
                        pv = -pv
                    idxbuf[i, pl.ds(k * L, L)] = bi
                    pvalbuf[i, pl.ds(k * L, L)] = pv
                    return 0
                lax.fori_loop(0, CHUNK // L, _vreg, 0)
                pltpu.async_copy(pvalbuf.at[i], tab_sh.at[idxbuf.at[i]], sem,
                                 add=True)

                @pl.when(i >= SCAT_WIN)
                def _():
                    pltpu.make_async_copy(
                        pvalbuf.at[i - SCAT_WIN],
                        tab_sh.at[idxbuf.at[i - SCAT_WIN]], sem).wait()
                return 0
            lax.fori_loop(0, NCHUNK, _chunk, 0)

            def _drain(k, _):
                i = NCHUNK - SCAT_WIN + k
                pltpu.make_async_copy(pvalbuf.at[i],
                                      tab_sh.at[idxbuf.at[i]], sem).wait()
                return 0
            lax.fori_loop(0, SCAT_WIN, _drain, 0)
        plsc.subcore_barrier()

        pltpu.sync_copy(tab_sh.at[pl.ds(s * BINS_PER_SUB, BINS_PER_SUB)],
                        t_chunk)

        def _csum(t, acc):
            v = t_chunk[pl.ds(t * L, L)]
            d = jnp.right_shift(v + (1 << 18), 19)
            return acc + d.astype(jnp.float32)
        chunk_sum_v = lax.fori_loop(0, BINS_PER_SUB // L, _csum,
                                    jnp.zeros((L,), jnp.float32))
        tt = jnp.sum(chunk_sum_v)
        tmp16[...] = jnp.where(iota > s, tt, 0.0)
        pltpu.sync_copy(tmp16, totals_sh.at[iota_r], add=True)
        plsc.subcore_barrier()
        pltpu.sync_copy(totals_sh, totv16)
        carry0 = jnp.sum(jnp.where(iota == s, totv16[...], 0.0))
        wq = wv * (1.0 / 1024.0)

        def _scan(t, st):
            carry, acc = st
            v = t_chunk[pl.ds(t * L, L)]
            di = jnp.right_shift(v + (1 << 18), 19)
            d = di.astype(jnp.float32)
            sv = (v - jnp.left_shift(di, 19)).astype(jnp.float32) * wq
            pc = plsc.cumsum(d)
            c_left = carry + (pc - d)
            acc = acc + jnp.abs(c_left * wv + sv)
            return (carry + jnp.sum(d), acc)
        _, accv = lax.fori_loop(0, BINS_PER_SUB // L, _scan,
                                (carry0, jnp.zeros((L,), jnp.float32)))
        partials[pl.ds(j * L, L)] = accv
        plsc.subcore_barrier()
        return 0

    lax.fori_loop(0, ROWS_PER_CORE, _row, 0)
    pltpu.sync_copy(partials, out_h.at[c, s])


def _sc_cramer(px, py, lo, w, invw):
    mesh = plsc.VectorSubcoreMesh(core_axis_name="c", subcore_axis_name="s")
    f = pl.kernel(
        _sc_body,
        out_type=jax.ShapeDtypeStruct((NC, NS, ROWS_PER_CORE * L), jnp.float32),
        mesh=mesh,
        scratch_types=[
            pltpu.VMEM((128,), jnp.float32),
            pltpu.VMEM((128,), jnp.float32),
            pltpu.VMEM((128,), jnp.float32),
            pltpu.VMEM((PTS_PER_SUB,), jnp.float32),
            pltpu.VMEM((NCHUNK, CHUNK), jnp.int32),
            pltpu.VMEM((NCHUNK, CHUNK), jnp.int32),
            pltpu.VMEM((BINS_PER_SUB,), jnp.int32),
            pltpu.VMEM((BINS_PER_SUB,), jnp.int32),
            pltpu.VMEM((L,), jnp.float32),
            pltpu.VMEM((L,), jnp.int32),
            pltpu.VMEM((L,), jnp.float32),
            pltpu.VMEM((ROWS_PER_CORE * L,), jnp.float32),
            pltpu.VMEM_SHARED((KBINS,), jnp.int32),
            pltpu.VMEM_SHARED((L,), jnp.float32),
            pltpu.SemaphoreType.DMA,
        ],
        compiler_params=pltpu.CompilerParams(needs_layout_passes=False),
    )
    return f(px, py, lo, w, invw)


def kernel(x, y, directions):
    b, ch, h, w_ = x.shape
    n = h * w_
    norm = jnp.sqrt(jnp.sum(jnp.square(directions), axis=-1, keepdims=True))
    dirs = directions / norm
    xr = x.reshape(ch, n)
    yr = y.reshape(ch, n)
    px, py, mn, mx = _project(dirs, xr, yr)
    lo = jnp.min(mn, axis=1)
    hi = jnp.max(mx, axis=1)
    wbin = jnp.maximum(hi - lo, 1e-30) / KBINS
    pad = jnp.zeros((128 - NDIR,), jnp.float32)
    lo_p = jnp.concatenate([lo, pad])
    w_p = jnp.concatenate([wbin, pad + 1.0])
    invw_p = jnp.concatenate([1.0 / wbin, pad + 1.0])
    out = _sc_cramer(px, py, lo_p, w_p, invw_p)
    return jnp.sum(out) / (NDIR * n)


# speedup vs baseline: 593.3129x; 1.0579x over previous
import functools

import jax
import jax.numpy as jnp
from jax import lax
from jax.experimental import pallas as pl
from jax.experimental.pallas import tpu as pltpu
from jax.experimental.pallas import tpu_sc as plsc

NDIR = 96
NPTS = 512 * 512
KBINS = 32768
NC, NS, L = 2, 16, 16
ROWS_PER_CORE = NDIR // NC
PTS_PER_SUB = NPTS // NS
CHUNK = 128
NCHUNK = PTS_PER_SUB // CHUNK
BINS_PER_SUB = KBINS // NS
BN = 2048
GRID = NPTS // BN
SCAT_WIN = 8


def _tc_body(dirs_ref, xr_ref, yr_ref, px_ref, py_ref, mn_ref, mx_ref):
    d = dirs_ref[...]
    pxb = jnp.dot(d, xr_ref[...], preferred_element_type=jnp.float32)
    pyb = jnp.dot(d, yr_ref[...], preferred_element_type=jnp.float32)
    px_ref[...] = pxb
    py_ref[...] = pyb
    bmin = jnp.minimum(jnp.min(pxb, axis=1, keepdims=True),
                       jnp.min(pyb, axis=1, keepdims=True))
    bmax = jnp.maximum(jnp.max(pxb, axis=1, keepdims=True),
                       jnp.max(pyb, axis=1, keepdims=True))
    bmin = jnp.broadcast_to(bmin, (NDIR, 128))
    bmax = jnp.broadcast_to(bmax, (NDIR, 128))
    j = pl.program_id(0)
    mn_ref[...] = jnp.where(j == 0, bmin, jnp.minimum(mn_ref[...], bmin))
    mx_ref[...] = jnp.where(j == 0, bmax, jnp.maximum(mx_ref[...], bmax))


def _project(dirs, xr, yr):
    return pl.pallas_call(
        _tc_body,
        grid=(GRID,),
        in_specs=[
            pl.BlockSpec((NDIR, NDIR), lambda j: (0, 0)),
            pl.BlockSpec((NDIR, BN), lambda j: (0, j)),
            pl.BlockSpec((NDIR, BN), lambda j: (0, j)),
        ],
        out_specs=[
            pl.BlockSpec((NDIR, BN), lambda j: (0, j)),
            pl.BlockSpec((NDIR, BN), lambda j: (0, j)),
            pl.BlockSpec((NDIR, 128), lambda j: (0, 0)),
            pl.BlockSpec((NDIR, 128), lambda j: (0, 0)),
        ],
        out_shape=[
            jax.ShapeDtypeStruct((NDIR, NPTS), jnp.float32),
            jax.ShapeDtypeStruct((NDIR, NPTS), jnp.float32),
            jax.ShapeDtypeStruct((NDIR, 128), jnp.float32),
            jax.ShapeDtypeStruct((NDIR, 128), jnp.float32),
        ],
        compiler_params=pltpu.CompilerParams(
            dimension_semantics=("arbitrary",)),
    )(dirs, xr, yr)


def _sc_body(px, py, lo_h, w_h, invw_h, out_h,
             lo_v, w_v, invw_v, ptu_buf, ptv_buf, idxbuf, pvalbuf,
             zeros_i, t_chunk, tmp16, iota_r, totv16, partials,
             taba_sh, tabb_sh, totals_sh, ssem, zsem, usem, vsem):
    c = lax.axis_index("c")
    s = lax.axis_index("s")
    iota = lax.iota(jnp.int32, L)
    my_bins = pl.ds(s * BINS_PER_SUB, BINS_PER_SUB)

    iota_r[...] = iota

    def _zinit(i, _):
        zeros_i[pl.ds(i * L, L)] = jnp.zeros((L,), jnp.int32)
        return 0
    lax.fori_loop(0, BINS_PER_SUB // L, _zinit, 0)

    pltpu.sync_copy(lo_h, lo_v)
    pltpu.sync_copy(w_h, w_v)
    pltpu.sync_copy(invw_h, invw_v)

    def _bcast(table_ref, r):
        acc = jnp.float32(0.0)
        for kk in range(128 // L):
            vec = table_ref[pl.ds(kk * L, L)]
            acc = acc + jnp.sum(jnp.where(iota + kk * L == r, vec, 0.0))
        return jnp.full((L,), acc, jnp.float32)

    def _prefetch(r):
        pltpu.async_copy(px.at[r, pl.ds(s * PTS_PER_SUB, PTS_PER_SUB)],
                         ptu_buf, usem)
        pltpu.async_copy(py.at[r, pl.ds(s * PTS_PER_SUB, PTS_PER_SUB)],
                         ptv_buf, vsem)

    def _do_row(j, tab_cur, tab_next):
        r = c * ROWS_PER_CORE + j
        lo = _bcast(lo_v, r)
        wv = _bcast(w_v, r)
        invw = _bcast(invw_v, r)

        @pl.when(j > 0)
        def _():
            pltpu.make_async_copy(zeros_i, tab_cur.at[my_bins], zsem).wait()
        plsc.subcore_barrier()

        for side in (0, 1):
            buf = ptu_buf if side == 0 else ptv_buf
            sem = usem if side == 0 else vsem
            pltpu.make_async_copy(
                (px if side == 0 else py).at[r, pl.ds(s * PTS_PER_SUB,
                                                      PTS_PER_SUB)],
                buf, sem).wait()

            def _chunk(i, _):
                def _vreg(k, _):
                    t = buf[pl.ds(i * CHUNK + k * L, L)]
                    f = (t - lo) * invw
                    bi = jnp.minimum(f, float(KBINS - 1)).astype(jnp.int32)
                    q = (((bi + 1).astype(jnp.float32) - f) * 1024.0
                         + 0.5).astype(jnp.int32)
                    pv = q + (1 << 19)
                    if side == 1:
                        pv = -pv
                    idxbuf[i, pl.ds(k * L, L)] = bi
                    pvalbuf[i, pl.ds(k * L, L)] = pv
                    return 0
                lax.fori_loop(0, CHUNK // L, _vreg, 0)
                pltpu.async_copy(pvalbuf.at[i], tab_cur.at[idxbuf.at[i]],
                                 ssem, add=True)

                @pl.when(i >= SCAT_WIN)
                def _():
                    pltpu.make_async_copy(
                        pvalbuf.at[i - SCAT_WIN],
                        tab_cur.at[idxbuf.at[i - SCAT_WIN]], ssem).wait()
                return 0
            lax.fori_loop(0, NCHUNK, _chunk, 0)

            def _drain(k, _):
                i = NCHUNK - SCAT_WIN + k
                pltpu.make_async_copy(pvalbuf.at[i],
                                      tab_cur.at[idxbuf.at[i]], ssem).wait()
                return 0
            lax.fori_loop(0, SCAT_WIN, _drain, 0)

        @pl.when(s == 0)
        def _():
            tmp16[...] = jnp.zeros((L,), jnp.float32)
            pltpu.sync_copy(tmp16, totals_sh)
        plsc.subcore_barrier()

        @pl.when(j < ROWS_PER_CORE - 1)
        def _():
            pltpu.async_copy(zeros_i, tab_next.at[my_bins], zsem)
            _prefetch(r + 1)

        pltpu.sync_copy(tab_cur.at[my_bins], t_chunk)

        def _csum(t, acc):
            v = t_chunk[pl.ds(t * L, L)]
            d = jnp.right_shift(v + (1 << 18), 19)
            return acc + d.astype(jnp.float32)
        chunk_sum_v = lax.fori_loop(0, BINS_PER_SUB // L, _csum,
                                    jnp.zeros((L,), jnp.float32))
        tt = jnp.sum(chunk_sum_v)
        tmp16[...] = jnp.where(iota > s, tt, 0.0)
        pltpu.sync_copy(tmp16, totals_sh.at[iota_r], add=True)
        plsc.subcore_barrier()
        pltpu.sync_copy(totals_sh, totv16)
        carry0 = jnp.sum(jnp.where(iota == s, totv16[...], 0.0))
        wq = wv * (1.0 / 1024.0)

        def _scan(t, st):
            carry, acc = st
            v = t_chunk[pl.ds(t * L, L)]
            di = jnp.right_shift(v + (1 << 18), 19)
            d = di.astype(jnp.float32)
            sv = (v - jnp.left_shift(di, 19)).astype(jnp.float32) * wq
            pc = plsc.cumsum(d)
            c_left = carry + (pc - d)
            acc = acc + jnp.abs(c_left * wv + sv)
            return (carry + jnp.sum(d), acc)
        _, accv = lax.fori_loop(0, BINS_PER_SUB // L, _scan,
                                (carry0, jnp.zeros((L,), jnp.float32)))
        partials[pl.ds(j * L, L)] = accv

    pltpu.sync_copy(zeros_i, taba_sh.at[my_bins])
    _prefetch(c * ROWS_PER_CORE)

    def _pair(p, _):
        _do_row(2 * p, taba_sh, tabb_sh)
        _do_row(2 * p + 1, tabb_sh, taba_sh)
        return 0
    lax.fori_loop(0, ROWS_PER_CORE // 2, _pair, 0)
    pltpu.sync_copy(partials, out_h.at[c, s])


def _sc_cramer(px, py, lo, w, invw):
    mesh = plsc.VectorSubcoreMesh(core_axis_name="c", subcore_axis_name="s")
    f = pl.kernel(
        _sc_body,
        out_type=jax.ShapeDtypeStruct((NC, NS, ROWS_PER_CORE * L), jnp.float32),
        mesh=mesh,
        scratch_types=[
            pltpu.VMEM((128,), jnp.float32),
            pltpu.VMEM((128,), jnp.float32),
            pltpu.VMEM((128,), jnp.float32),
            pltpu.VMEM((PTS_PER_SUB,), jnp.float32),
            pltpu.VMEM((PTS_PER_SUB,), jnp.float32),
            pltpu.VMEM((NCHUNK, CHUNK), jnp.int32),
            pltpu.VMEM((NCHUNK, CHUNK), jnp.int32),
            pltpu.VMEM((BINS_PER_SUB,), jnp.int32),
            pltpu.VMEM((BINS_PER_SUB,), jnp.int32),
            pltpu.VMEM((L,), jnp.float32),
            pltpu.VMEM((L,), jnp.int32),
            pltpu.VMEM((L,), jnp.float32),
            pltpu.VMEM((ROWS_PER_CORE * L,), jnp.float32),
            pltpu.VMEM_SHARED((KBINS,), jnp.int32),
            pltpu.VMEM_SHARED((KBINS,), jnp.int32),
            pltpu.VMEM_SHARED((L,), jnp.float32),
            pltpu.SemaphoreType.DMA,
            pltpu.SemaphoreType.DMA,
            pltpu.SemaphoreType.DMA,
            pltpu.SemaphoreType.DMA,
        ],
        compiler_params=pltpu.CompilerParams(needs_layout_passes=False),
    )
    return f(px, py, lo, w, invw)


def kernel(x, y, directions):
    b, ch, h, w_ = x.shape
    n = h * w_
    norm = jnp.sqrt(jnp.sum(jnp.square(directions), axis=-1, keepdims=True))
    dirs = directions / norm
    xr = x.reshape(ch, n)
    yr = y.reshape(ch, n)
    px, py, mn, mx = _project(dirs, xr, yr)
    lo = jnp.min(mn, axis=1)
    hi = jnp.max(mx, axis=1)
    wbin = jnp.maximum(hi - lo, 1e-30) / KBINS
    pad = jnp.zeros((128 - NDIR,), jnp.float32)
    lo_p = jnp.concatenate([lo, pad])
    w_p = jnp.concatenate([wbin, pad + 1.0])
    invw_p = jnp.concatenate([1.0 / wbin, pad + 1.0])
    out = _sc_cramer(px, py, lo_p, w_p, invw_p)
    return jnp.sum(out) / (NDIR * n)


# speedup vs baseline: 595.0024x; 1.0028x over previous
import functools

import jax
import jax.numpy as jnp
from jax import lax
from jax.experimental import pallas as pl
from jax.experimental.pallas import tpu as pltpu
from jax.experimental.pallas import tpu_sc as plsc

NDIR = 96
NPTS = 512 * 512
KBINS = 32768
NC, NS, L = 2, 16, 16
ROWS_PER_CORE = NDIR // NC // 2
PTS_PER_SUB = NPTS // NS
CHUNK = 128
NCHUNK = PTS_PER_SUB // CHUNK
BINS_PER_SUB = KBINS // NS
BN = 2048
GRID = NPTS // BN
SCAT_WIN = 8


def _tc_body(dirs_ref, xr_ref, yr_ref, px_ref, py_ref, mn_ref, mx_ref):
    d = dirs_ref[...]
    pxb = jnp.dot(d, xr_ref[...], preferred_element_type=jnp.float32)
    pyb = jnp.dot(d, yr_ref[...], preferred_element_type=jnp.float32)
    px_ref[...] = pxb
    py_ref[...] = pyb
    bmin = jnp.minimum(jnp.min(pxb, axis=1, keepdims=True),
                       jnp.min(pyb, axis=1, keepdims=True))
    bmax = jnp.maximum(jnp.max(pxb, axis=1, keepdims=True),
                       jnp.max(pyb, axis=1, keepdims=True))
    bmin = jnp.broadcast_to(bmin, (bmin.shape[0], 128))
    bmax = jnp.broadcast_to(bmax, (bmax.shape[0], 128))
    j = pl.program_id(0)
    mn_ref[...] = jnp.where(j == 0, bmin, jnp.minimum(mn_ref[...], bmin))
    mx_ref[...] = jnp.where(j == 0, bmax, jnp.maximum(mx_ref[...], bmax))


def _project(dirs, xr, yr):
    nd = dirs.shape[0]
    return pl.pallas_call(
        _tc_body,
        grid=(GRID,),
        in_specs=[
            pl.BlockSpec((nd, NDIR), lambda j: (0, 0)),
            pl.BlockSpec((NDIR, BN), lambda j: (0, j)),
            pl.BlockSpec((NDIR, BN), lambda j: (0, j)),
        ],
        out_specs=[
            pl.BlockSpec((nd, BN), lambda j: (0, j)),
            pl.BlockSpec((nd, BN), lambda j: (0, j)),
            pl.BlockSpec((nd, 128), lambda j: (0, 0)),
            pl.BlockSpec((nd, 128), lambda j: (0, 0)),
        ],
        out_shape=[
            jax.ShapeDtypeStruct((nd, NPTS), jnp.float32),
            jax.ShapeDtypeStruct((nd, NPTS), jnp.float32),
            jax.ShapeDtypeStruct((nd, 128), jnp.float32),
            jax.ShapeDtypeStruct((nd, 128), jnp.float32),
        ],
        compiler_params=pltpu.CompilerParams(
            dimension_semantics=("arbitrary",)),
    )(dirs, xr, yr)


def _sc_body(px, py, lo_h, w_h, invw_h, out_h,
             lo_v, w_v, invw_v, ptu_buf, ptv_buf, idxbuf, pvalbuf,
             zeros_i, t_chunk, tmp16, iota_r, totv16, partials,
             taba_sh, tabb_sh, totals_sh, ssem, zsem, usem, vsem):
    c = lax.axis_index("c")
    s = lax.axis_index("s")
    iota = lax.iota(jnp.int32, L)
    my_bins = pl.ds(s * BINS_PER_SUB, BINS_PER_SUB)

    iota_r[...] = iota

    def _zinit(i, _):
        zeros_i[pl.ds(i * L, L)] = jnp.zeros((L,), jnp.int32)
        return 0
    lax.fori_loop(0, BINS_PER_SUB // L, _zinit, 0)

    pltpu.sync_copy(lo_h, lo_v)
    pltpu.sync_copy(w_h, w_v)
    pltpu.sync_copy(invw_h, invw_v)

    def _bcast(table_ref, r):
        acc = jnp.float32(0.0)
        for kk in range(128 // L):
            vec = table_ref[pl.ds(kk * L, L)]
            acc = acc + jnp.sum(jnp.where(iota + kk * L == r, vec, 0.0))
        return jnp.full((L,), acc, jnp.float32)

    def _prefetch(r):
        pltpu.async_copy(px.at[r, pl.ds(s * PTS_PER_SUB, PTS_PER_SUB)],
                         ptu_buf, usem)
        pltpu.async_copy(py.at[r, pl.ds(s * PTS_PER_SUB, PTS_PER_SUB)],
                         ptv_buf, vsem)

    def _do_row(j, tab_cur, tab_next):
        r = c * ROWS_PER_CORE + j
        lo = _bcast(lo_v, r)
        wv = _bcast(w_v, r)
        invw = _bcast(invw_v, r)

        @pl.when(j > 0)
        def _():
            pltpu.make_async_copy(zeros_i, tab_cur.at[my_bins], zsem).wait()
        plsc.subcore_barrier()

        for side in (0, 1):
            buf = ptu_buf if side == 0 else ptv_buf
            sem = usem if side == 0 else vsem
            pltpu.make_async_copy(
                (px if side == 0 else py).at[r, pl.ds(s * PTS_PER_SUB,
                                                      PTS_PER_SUB)],
                buf, sem).wait()

            def _chunk(i, _):
                def _vreg(k, _):
                    t = buf[pl.ds(i * CHUNK + k * L, L)]
                    f = (t - lo) * invw
                    bi = jnp.minimum(f, float(KBINS - 1)).astype(jnp.int32)
                    q = (((bi + 1).astype(jnp.float32) - f) * 1024.0
                         + 0.5).astype(jnp.int32)
                    pv = q + (1 << 19)
                    if side == 1:
                        pv = -pv
                    idxbuf[i, pl.ds(k * L, L)] = bi
                    pvalbuf[i, pl.ds(k * L, L)] = pv
                    return 0
                lax.fori_loop(0, CHUNK // L, _vreg, 0)
                pltpu.async_copy(pvalbuf.at[i], tab_cur.at[idxbuf.at[i]],
                                 ssem, add=True)

                @pl.when(i >= SCAT_WIN)
                def _():
                    pltpu.make_async_copy(
                        pvalbuf.at[i - SCAT_WIN],
                        tab_cur.at[idxbuf.at[i - SCAT_WIN]], ssem).wait()
                return 0
            lax.fori_loop(0, NCHUNK, _chunk, 0)

            def _drain(k, _):
                i = NCHUNK - SCAT_WIN + k
                pltpu.make_async_copy(pvalbuf.at[i],
                                      tab_cur.at[idxbuf.at[i]], ssem).wait()
                return 0
            lax.fori_loop(0, SCAT_WIN, _drain, 0)

        @pl.when(s == 0)
        def _():
            tmp16[...] = jnp.zeros((L,), jnp.float32)
            pltpu.sync_copy(tmp16, totals_sh)
        plsc.subcore_barrier()

        @pl.when(j < ROWS_PER_CORE - 1)
        def _():
            pltpu.async_copy(zeros_i, tab_next.at[my_bins], zsem)
            _prefetch(r + 1)

        pltpu.sync_copy(tab_cur.at[my_bins], t_chunk)

        def _csum(t, acc):
            v = t_chunk[pl.ds(t * L, L)]
            d = jnp.right_shift(v + (1 << 18), 19)
            return acc + d.astype(jnp.float32)
        chunk_sum_v = lax.fori_loop(0, BINS_PER_SUB // L, _csum,
                                    jnp.zeros((L,), jnp.float32))
        tt = jnp.sum(chunk_sum_v)
        tmp16[...] = jnp.where(iota > s, tt, 0.0)
        pltpu.sync_copy(tmp16, totals_sh.at[iota_r], add=True)
        plsc.subcore_barrier()
        pltpu.sync_copy(totals_sh, totv16)
        carry0 = jnp.sum(jnp.where(iota == s, totv16[...], 0.0))
        wq = wv * (1.0 / 1024.0)

        def _scan(t, st):
            carry, acc = st
            v = t_chunk[pl.ds(t * L, L)]
            di = jnp.right_shift(v + (1 << 18), 19)
            d = di.astype(jnp.float32)
            sv = (v - jnp.left_shift(di, 19)).astype(jnp.float32) * wq
            pc = plsc.cumsum(d)
            c_left = carry + (pc - d)
            acc = acc + jnp.abs(c_left * wv + sv)
            return (carry + jnp.sum(d), acc)
        _, accv = lax.fori_loop(0, BINS_PER_SUB // L, _scan,
                                (carry0, jnp.zeros((L,), jnp.float32)))
        partials[pl.ds(j * L, L)] = accv

    pltpu.sync_copy(zeros_i, taba_sh.at[my_bins])
    _prefetch(c * ROWS_PER_CORE)

    def _pair(p, _):
        _do_row(2 * p, taba_sh, tabb_sh)
        _do_row(2 * p + 1, tabb_sh, taba_sh)
        return 0
    lax.fori_loop(0, ROWS_PER_CORE // 2, _pair, 0)
    pltpu.sync_copy(partials, out_h.at[c, s])


def _sc_cramer(px, py, lo, w, invw):
    mesh = plsc.VectorSubcoreMesh(core_axis_name="c", subcore_axis_name="s")
    f = pl.kernel(
        _sc_body,
        out_type=jax.ShapeDtypeStruct((NC, NS, ROWS_PER_CORE * L), jnp.float32),
        mesh=mesh,
        scratch_types=[
            pltpu.VMEM((128,), jnp.float32),
            pltpu.VMEM((128,), jnp.float32),
            pltpu.VMEM((128,), jnp.float32),
            pltpu.VMEM((PTS_PER_SUB,), jnp.float32),
            pltpu.VMEM((PTS_PER_SUB,), jnp.float32),
            pltpu.VMEM((NCHUNK, CHUNK), jnp.int32),
            pltpu.VMEM((NCHUNK, CHUNK), jnp.int32),
            pltpu.VMEM((BINS_PER_SUB,), jnp.int32),
            pltpu.VMEM((BINS_PER_SUB,), jnp.int32),
            pltpu.VMEM((L,), jnp.float32),
            pltpu.VMEM((L,), jnp.int32),
            pltpu.VMEM((L,), jnp.float32),
            pltpu.VMEM((ROWS_PER_CORE * L,), jnp.float32),
            pltpu.VMEM_SHARED((KBINS,), jnp.int32),
            pltpu.VMEM_SHARED((KBINS,), jnp.int32),
            pltpu.VMEM_SHARED((L,), jnp.float32),
            pltpu.SemaphoreType.DMA,
            pltpu.SemaphoreType.DMA,
            pltpu.SemaphoreType.DMA,
            pltpu.SemaphoreType.DMA,
        ],
        compiler_params=pltpu.CompilerParams(needs_layout_passes=False),
    )
    return f(px, py, lo, w, invw)


def kernel(x, y, directions):
    b, ch, h, w_ = x.shape
    n = h * w_
    norm = jnp.sqrt(jnp.sum(jnp.square(directions), axis=-1, keepdims=True))
    dirs = directions / norm
    xr = x.reshape(ch, n)
    yr = y.reshape(ch, n)
    half = NDIR // 2
    total = jnp.float32(0.0)
    pad = jnp.zeros((128 - half,), jnp.float32)
    for hblk in range(2):
        d_h = lax.slice_in_dim(dirs, hblk * half, (hblk + 1) * half, axis=0)
        pxh, pyh, mnh, mxh = _project(d_h, xr, yr)
        lo = jnp.min(mnh, axis=1)
        hi = jnp.max(mxh, axis=1)
        wbin = jnp.maximum(hi - lo, 1e-30) / KBINS
        lo_p = jnp.concatenate([lo, pad])
        w_p = jnp.concatenate([wbin, pad + 1.0])
        invw_p = jnp.concatenate([1.0 / wbin, pad + 1.0])
        out = _sc_cramer(pxh, pyh, lo_p, w_p, invw_p)
        total = total + jnp.sum(out)
    return total / (NDIR * n)


# speedup vs baseline: 595.4048x; 1.0007x over previous
import jax
import jax.numpy as jnp
from jax import lax
from jax.experimental import pallas as pl
from jax.experimental.pallas import tpu as pltpu
from jax.experimental.pallas import tpu_sc as plsc

NDIR = 96
NPTS = 512 * 512
KBINS = 32768
NC, NS, L = 2, 16, 16
ROWS_PER_CORE = NDIR // NC // 2
PTS_PER_SUB = NPTS // NS
CHUNK = 128
NCHUNK = PTS_PER_SUB // CHUNK
BINS_PER_SUB = KBINS // NS
BN = 2048
GRID = NPTS // BN
SCAT_WIN = 8


def _tc_body(dirs_ref, xr_ref, yr_ref, px_ref, py_ref, mn_ref, mx_ref):
    d = dirs_ref[...]
    pxb = jnp.dot(d, xr_ref[...], preferred_element_type=jnp.float32)
    pyb = jnp.dot(d, yr_ref[...], preferred_element_type=jnp.float32)
    px_ref[...] = pxb
    py_ref[...] = pyb
    bmin = jnp.minimum(jnp.min(pxb, axis=1, keepdims=True),
                       jnp.min(pyb, axis=1, keepdims=True))
    bmax = jnp.maximum(jnp.max(pxb, axis=1, keepdims=True),
                       jnp.max(pyb, axis=1, keepdims=True))
    bmin = jnp.broadcast_to(bmin, (bmin.shape[0], 128))
    bmax = jnp.broadcast_to(bmax, (bmax.shape[0], 128))
    j = pl.program_id(0)
    mn_ref[...] = jnp.where(j == 0, bmin, jnp.minimum(mn_ref[...], bmin))
    mx_ref[...] = jnp.where(j == 0, bmax, jnp.maximum(mx_ref[...], bmax))


def _project(dirs, xr, yr):
    nd = dirs.shape[0]
    return pl.pallas_call(
        _tc_body,
        grid=(GRID,),
        in_specs=[
            pl.BlockSpec((nd, NDIR), lambda j: (0, 0)),
            pl.BlockSpec((NDIR, BN), lambda j: (0, j)),
            pl.BlockSpec((NDIR, BN), lambda j: (0, j)),
        ],
        out_specs=[
            pl.BlockSpec((nd, BN), lambda j: (0, j)),
            pl.BlockSpec((nd, BN), lambda j: (0, j)),
            pl.BlockSpec((nd, 128), lambda j: (0, 0)),
            pl.BlockSpec((nd, 128), lambda j: (0, 0)),
        ],
        out_shape=[
            jax.ShapeDtypeStruct((nd, NPTS), jnp.float32),
            jax.ShapeDtypeStruct((nd, NPTS), jnp.float32),
            jax.ShapeDtypeStruct((nd, 128), jnp.float32),
            jax.ShapeDtypeStruct((nd, 128), jnp.float32),
        ],
        compiler_params=pltpu.CompilerParams(
            dimension_semantics=("arbitrary",)),
    )(dirs, xr, yr)


def _sc_body(px, py, lo_h, w_h, invw_h, out_h,
             lo_v, w_v, invw_v, ptu_buf, ptv_buf, idxbuf, pvalbuf,
             zeros_i, t_chunk, tmp16, iota_r, totv16, partials,
             taba_sh, tabb_sh, totals_sh, ssem, zsem, usem, vsem):
    c = lax.axis_index("c")
    s = lax.axis_index("s")
    iota = lax.iota(jnp.int32, L)
    my_bins = pl.ds(s * BINS_PER_SUB, BINS_PER_SUB)

    iota_r[...] = iota

    def _zinit(i, _):
        zeros_i[pl.ds(i * L, L)] = jnp.zeros((L,), jnp.int32)
        return 0
    lax.fori_loop(0, BINS_PER_SUB // L, _zinit, 0)

    pltpu.sync_copy(lo_h, lo_v)
    pltpu.sync_copy(w_h, w_v)
    pltpu.sync_copy(invw_h, invw_v)

    def _bcast(table_ref, r):
        acc = jnp.float32(0.0)
        for kk in range(128 // L):
            vec = table_ref[pl.ds(kk * L, L)]
            acc = acc + jnp.sum(jnp.where(iota + kk * L == r, vec, 0.0))
        return jnp.full((L,), acc, jnp.float32)

    def _prefetch(r):
        pltpu.async_copy(px.at[r, pl.ds(s * PTS_PER_SUB, PTS_PER_SUB)],
                         ptu_buf, usem)
        pltpu.async_copy(py.at[r, pl.ds(s * PTS_PER_SUB, PTS_PER_SUB)],
                         ptv_buf, vsem)

    def _do_row(j, tab_cur, tab_next):
        r = c * ROWS_PER_CORE + j
        lo = _bcast(lo_v, r)
        wv = _bcast(w_v, r)
        invw = _bcast(invw_v, r)

        @pl.when(j > 0)
        def _():
            pltpu.make_async_copy(zeros_i, tab_cur.at[my_bins], zsem).wait()
        plsc.subcore_barrier()

        for side in (0, 1):
            buf = ptu_buf if side == 0 else ptv_buf
            sem = usem if side == 0 else vsem
            pltpu.make_async_copy(
                (px if side == 0 else py).at[r, pl.ds(s * PTS_PER_SUB,
                                                      PTS_PER_SUB)],
                buf, sem).wait()

            def _chunk(i, _):
                def _vreg(k, _):
                    t = buf[pl.ds(i * CHUNK + k * L, L)]
                    f = (t - lo) * invw
                    bi = jnp.minimum(f, float(KBINS - 1)).astype(jnp.int32)
                    q = (((bi + 1).astype(jnp.float32) - f) * 1024.0
                         + 0.5).astype(jnp.int32)
                    pv = q + (1 << 19)
                    if side == 1:
                        pv = -pv
                    idxbuf[i, pl.ds(k * L, L)] = bi
                    pvalbuf[i, pl.ds(k * L, L)] = pv
                    return 0
                lax.fori_loop(0, CHUNK // L, _vreg, 0)
                pltpu.async_copy(pvalbuf.at[i], tab_cur.at[idxbuf.at[i]],
                                 ssem, add=True)

                @pl.when(i >= SCAT_WIN)
                def _():
                    pltpu.make_async_copy(
                        pvalbuf.at[i - SCAT_WIN],
                        tab_cur.at[idxbuf.at[i - SCAT_WIN]], ssem).wait()
                return 0
            lax.fori_loop(0, NCHUNK, _chunk, 0)

            def _drain(k, _):
                i = NCHUNK - SCAT_WIN + k
                pltpu.make_async_copy(pvalbuf.at[i],
                                      tab_cur.at[idxbuf.at[i]], ssem).wait()
                return 0
            lax.fori_loop(0, SCAT_WIN, _drain, 0)

        @pl.when(s == 0)
        def _():
            tmp16[...] = jnp.zeros((L,), jnp.float32)
            pltpu.sync_copy(tmp16, totals_sh)
        plsc.subcore_barrier()

        @pl.when(j < ROWS_PER_CORE - 1)
        def _():
            pltpu.async_copy(zeros_i, tab_next.at[my_bins], zsem)
            _prefetch(r + 1)

        pltpu.sync_copy(tab_cur.at[my_bins], t_chunk)

        def _csum(t, acc):
            v = t_chunk[pl.ds(t * L, L)]
            d = jnp.right_shift(v + (1 << 18), 19)
            return acc + d.astype(jnp.float32)
        chunk_sum_v = lax.fori_loop(0, BINS_PER_SUB // L, _csum,
                                    jnp.zeros((L,), jnp.float32))
        tt = jnp.sum(chunk_sum_v)
        tmp16[...] = jnp.where(iota > s, tt, 0.0)
        pltpu.sync_copy(tmp16, totals_sh.at[iota_r], add=True)
        plsc.subcore_barrier()
        pltpu.sync_copy(totals_sh, totv16)
        carry0 = jnp.sum(jnp.where(iota == s, totv16[...], 0.0))
        wq = wv * (1.0 / 1024.0)

        def _scan(t, st):
            carry, acc = st
            v = t_chunk[pl.ds(t * L, L)]
            di = jnp.right_shift(v + (1 << 18), 19)
            d = di.astype(jnp.float32)
            sv = (v - jnp.left_shift(di, 19)).astype(jnp.float32) * wq
            pc = plsc.cumsum(d)
            c_left = carry + (pc - d)
            acc = acc + jnp.abs(c_left * wv + sv)
            return (carry + jnp.sum(d), acc)
        _, accv = lax.fori_loop(0, BINS_PER_SUB // L, _scan,
                                (carry0, jnp.zeros((L,), jnp.float32)))
        partials[pl.ds(j * L, L)] = accv

    pltpu.sync_copy(zeros_i, taba_sh.at[my_bins])
    _prefetch(c * ROWS_PER_CORE)

    def _pair(p, _):
        _do_row(2 * p, taba_sh, tabb_sh)
        _do_row(2 * p + 1, tabb_sh, taba_sh)
        return 0
    lax.fori_loop(0, ROWS_PER_CORE // 2, _pair, 0)
    pltpu.sync_copy(partials, out_h.at[c, s])


def _sc_cramer(px, py, lo, w, invw):
    mesh = plsc.VectorSubcoreMesh(core_axis_name="c", subcore_axis_name="s")
    f = pl.kernel(
        _sc_body,
        out_type=jax.ShapeDtypeStruct((NC, NS, ROWS_PER_CORE * L), jnp.float32),
        mesh=mesh,
        scratch_types=[
            pltpu.VMEM((128,), jnp.float32),
            pltpu.VMEM((128,), jnp.float32),
            pltpu.VMEM((128,), jnp.float32),
            pltpu.VMEM((PTS_PER_SUB,), jnp.float32),
            pltpu.VMEM((PTS_PER_SUB,), jnp.float32),
            pltpu.VMEM((NCHUNK, CHUNK), jnp.int32),
            pltpu.VMEM((NCHUNK, CHUNK), jnp.int32),
            pltpu.VMEM((BINS_PER_SUB,), jnp.int32),
            pltpu.VMEM((BINS_PER_SUB,), jnp.int32),
            pltpu.VMEM((L,), jnp.float32),
            pltpu.VMEM((L,), jnp.int32),
            pltpu.VMEM((L,), jnp.float32),
            pltpu.VMEM((ROWS_PER_CORE * L,), jnp.float32),
            pltpu.VMEM_SHARED((KBINS,), jnp.int32),
            pltpu.VMEM_SHARED((KBINS,), jnp.int32),
            pltpu.VMEM_SHARED((L,), jnp.float32),
            pltpu.SemaphoreType.DMA,
            pltpu.SemaphoreType.DMA,
            pltpu.SemaphoreType.DMA,
            pltpu.SemaphoreType.DMA,
        ],
        compiler_params=pltpu.CompilerParams(needs_layout_passes=False),
    )
    return f(px, py, lo, w, invw)


def kernel(x, y, directions):
    b, ch, h, w_ = x.shape
    n = h * w_
    norm = jnp.sqrt(jnp.sum(jnp.square(directions), axis=-1, keepdims=True))
    dirs = directions / norm
    xr = x.reshape(ch, n)
    yr = y.reshape(ch, n)
    half = NDIR // 2
    total = jnp.float32(0.0)
    pad = jnp.zeros((128 - half,), jnp.float32)
    for hblk in range(2):
        d_h = lax.slice_in_dim(dirs, hblk * half, (hblk + 1) * half, axis=0)
        pxh, pyh, mnh, mxh = _project(d_h, xr, yr)
        lo = jnp.min(mnh, axis=1)
        hi = jnp.max(mxh, axis=1)
        wbin = jnp.maximum(hi - lo, 1e-30) / KBINS
        lo_p = jnp.concatenate([lo, pad])
        w_p = jnp.concatenate([wbin, pad + 1.0])
        invw_p = jnp.concatenate([1.0 / wbin, pad + 1.0])
        out = _sc_cramer(pxh, pyh, lo_p, w_p, invw_p)
        total = total + jnp.sum(out)
    return total / (NDIR * n)
